# baseline (speedup 1.0000x reference)
import jax
import jax.numpy as jnp
from jax import lax
from jax.experimental import pallas as pl
from jax.experimental.pallas import tpu as pltpu

_B = 4
_S = 2048
_D = 4096


def _tc_body(ids_ref, hidden_hbm, out_ref, sem):
    b = pl.program_id(0)
    col = lax.broadcasted_iota(jnp.int32, (1, 1, _S), 2)
    key = ids_ref[...] * _S + ((_S - 1) - col)
    best = jnp.max(key)
    idx = (_S - 1) - (best & (_S - 1))
    pltpu.make_async_copy(
        hidden_hbm.at[pl.ds(b * _S + idx, 1), :],
        out_ref.at[pl.ds(b, 1), :],
        sem.at[b],
    ).start()

    @pl.when(b == _B - 1)
    def _():
        for j in range(_B):
            pltpu.make_async_copy(
                hidden_hbm.at[pl.ds(0, 1), :],
                out_ref.at[pl.ds(j, 1), :],
                sem.at[j],
            ).wait()


@jax.jit
def kernel(last_hidden_state, input_ids):
    hidden2d = last_hidden_state.reshape(_B * _S, _D)
    ids3 = input_ids.reshape(_B, 1, _S)
    return pl.pallas_call(
        _tc_body,
        grid=(_B,),
        out_shape=jax.ShapeDtypeStruct((_B, _D), jnp.float32),
        in_specs=[
            pl.BlockSpec((1, 1, _S), lambda b: (b, 0, 0)),
            pl.BlockSpec(memory_space=pltpu.MemorySpace.HBM),
        ],
        out_specs=pl.BlockSpec((_B, _D), lambda b: (0, 0)),
        scratch_shapes=[pltpu.SemaphoreType.DMA((_B,))],
    )(ids3, hidden2d)


# speedup vs baseline: 2.0091x; 2.0091x over previous
import jax
import jax.numpy as jnp
from jax import lax
from jax.experimental import pallas as pl
from jax.experimental.pallas import tpu as pltpu

_B = 4
_S = 2048
_D = 4096


def _tc_body(ids_ref, hidden_hbm, out_ref, sem):
    col = lax.broadcasted_iota(jnp.int32, (_B, _S), 1)
    key = ids_ref[...] * _S + ((_S - 1) - col)
    for b in range(_B):
        best = jnp.max(key[b : b + 1, :])
        idx = (_S - 1) - (best & (_S - 1))
        pltpu.make_async_copy(
            hidden_hbm.at[pl.ds(b * _S + idx, 1), :],
            out_ref.at[pl.ds(b, 1), :],
            sem,
        ).start()
    pltpu.make_async_copy(hidden_hbm.at[pl.ds(0, _B), :], out_ref, sem).wait()


@jax.jit
def kernel(last_hidden_state, input_ids):
    hidden2d = last_hidden_state.reshape(_B * _S, _D)
    return pl.pallas_call(
        _tc_body,
        out_shape=jax.ShapeDtypeStruct((_B, _D), jnp.float32),
        in_specs=[
            pl.BlockSpec(memory_space=pltpu.VMEM),
            pl.BlockSpec(memory_space=pltpu.MemorySpace.HBM),
        ],
        out_specs=pl.BlockSpec(memory_space=pltpu.VMEM),
        scratch_shapes=[pltpu.SemaphoreType.DMA],
    )(input_ids, hidden2d)
